# baseline (speedup 1.0000x reference)
import functools

import jax
import jax.numpy as jnp
from jax import lax
from jax.experimental import pallas as pl
from jax.experimental.pallas import tpu as pltpu
from jax.experimental.pallas import tpu_sc as plsc

D_WORD = 64
D_FEAT = 16
D_OUT = 96
D_PAD = 128
CHUNK = 256


def _embed(emb_lut, comb_table, src, cidx):
    N = src.shape[0]
    info = plsc.get_sparse_core_info()
    NC, NS = info.num_cores, info.num_subcores
    NW = NC * NS
    assert N % NW == 0
    tok_per_w = N // NW
    assert tok_per_w % CHUNK == 0
    n_chunks = tok_per_w // CHUNK

    mesh = plsc.VectorSubcoreMesh(core_axis_name="c", subcore_axis_name="s")

    @functools.partial(
        pl.kernel,
        out_type=jax.ShapeDtypeStruct((N, D_PAD), jnp.float32),
        mesh=mesh,
        compiler_params=pltpu.CompilerParams(use_tc_tiling_on_sc=False),
        scratch_types=[
            [pltpu.VMEM((CHUNK,), jnp.int32) for _ in range(2)],
            [pltpu.VMEM((CHUNK,), jnp.int32) for _ in range(2)],
            [pltpu.VMEM((CHUNK, D_WORD), jnp.float32) for _ in range(2)],
            [pltpu.VMEM((CHUNK, 2 * D_FEAT), jnp.float32) for _ in range(2)],
            [pltpu.VMEM((CHUNK, D_PAD), jnp.float32) for _ in range(2)],
            [pltpu.SemaphoreType.DMA for _ in range(2)],
            [pltpu.SemaphoreType.DMA for _ in range(2)],
            [pltpu.SemaphoreType.DMA for _ in range(2)],
        ],
    )
    def body(emb_hbm, comb_hbm, src_hbm, cidx_hbm, out_hbm,
             wi, ci, wbuf, cbuf, obuf, si, sg, so):
        wid = lax.axis_index("s") * NC + lax.axis_index("c")
        base0 = wid * tok_per_w

        def idx_copies(c, s):
            base = base0 + c * CHUNK
            return (
                pltpu.make_async_copy(src_hbm.at[pl.ds(base, CHUNK)], wi[s], si[s]),
                pltpu.make_async_copy(cidx_hbm.at[pl.ds(base, CHUNK)], ci[s], si[s]),
            )

        def gather_copies(s):
            return (
                pltpu.make_async_copy(emb_hbm.at[wi[s]], wbuf[s], sg[s]),
                pltpu.make_async_copy(comb_hbm.at[ci[s]], cbuf[s], sg[s]),
            )

        def out_copy(c, s):
            base = base0 + c * CHUNK
            return pltpu.make_async_copy(obuf[s], out_hbm.at[pl.ds(base, CHUNK)], so[s])

        def start(c, s):
            for cp in idx_copies(c, s):
                cp.start()

        def mid(c, s):
            for cp in idx_copies(c, s):
                cp.wait()
            for cp in gather_copies(s):
                cp.start()

        UNROLL = 8

        def assemble_one(s):
            def assemble(g, carry):
                j0 = g * UNROLL
                for u in range(UNROLL):
                    j = j0 + u
                    for k in range(D_WORD // 16):
                        obuf[s][j, pl.ds(16 * k, 16)] = wbuf[s][j, pl.ds(16 * k, 16)]
                    obuf[s][j, pl.ds(D_WORD, 16)] = cbuf[s][j, pl.ds(0, 16)]
                    obuf[s][j, pl.ds(D_WORD + D_FEAT, 16)] = cbuf[s][j, pl.ds(D_FEAT, 16)]
                return carry

            lax.fori_loop(0, CHUNK // UNROLL, assemble, 0)

        def step(i, b):
            mid(i + 1, 1 - b)
            for cp in gather_copies(b):
                cp.wait()

            @pl.when(i < n_chunks - 2)
            def _():
                start(i + 2, b)

            @pl.when(i >= 2)
            def _():
                out_copy(i, b).wait()

            assemble_one(b)
            out_copy(i, b).start()

        assert n_chunks % 2 == 0 and n_chunks >= 4

        start(0, 0)
        start(1, 1)
        mid(0, 0)

        def pair(p, carry):
            for b in range(2):
                step(2 * p + b, b)
            return carry

        lax.fori_loop(0, (n_chunks - 2) // 2, pair, 0)

        step(n_chunks - 2, 0)

        c = n_chunks - 1
        for cp in gather_copies(1):
            cp.wait()
        out_copy(c, 1).wait()
        assemble_one(1)
        out_copy(c, 1).start()
        out_copy(c, 1).wait()
        out_copy(c - 1, 0).wait()

    return body(emb_lut, comb_table, src, cidx)


def kernel(emb_lut, pos_table, ner_table, source, pos_idx, ner_idx):
    B, L = source.shape
    N = B * L
    V = emb_lut.shape[0]
    n_ner = ner_table.shape[0]
    emb2 = jnp.pad(emb_lut[:, None, :], ((0, 0), (0, 1), (0, 0))).reshape(2 * V, D_WORD)
    src = source.reshape(N).astype(jnp.int32) * 2
    cidx = pos_idx.reshape(N).astype(jnp.int32) * n_ner + ner_idx.reshape(N).astype(jnp.int32)
    comb = jnp.concatenate(
        [jnp.repeat(pos_table, n_ner, axis=0), jnp.tile(ner_table, (pos_table.shape[0], 1))],
        axis=1,
    )
    out = _embed(emb2, comb, src, cidx)
    return out[:, :D_OUT].reshape(B, L, D_OUT)


# speedup vs baseline: 1.9910x; 1.9910x over previous
import functools

import jax
import jax.numpy as jnp
from jax import lax
from jax.experimental import pallas as pl
from jax.experimental.pallas import tpu as pltpu
from jax.experimental.pallas import tpu_sc as plsc

D_WORD = 64
D_FEAT = 16
D_OUT = 96
D_PAD = 128
CHUNK = 128


def _embed(emb_lut, comb_table, src, cidx):
    N = src.shape[0]
    info = plsc.get_sparse_core_info()
    NC, NS = info.num_cores, info.num_subcores
    NW = NC * NS
    assert N % NW == 0
    tok_per_w = N // NW
    assert tok_per_w % CHUNK == 0
    n_chunks = tok_per_w // CHUNK

    mesh = plsc.VectorSubcoreMesh(core_axis_name="c", subcore_axis_name="s")

    @functools.partial(
        pl.kernel,
        out_type=jax.ShapeDtypeStruct((N, D_PAD), jnp.float32),
        mesh=mesh,
        compiler_params=pltpu.CompilerParams(use_tc_tiling_on_sc=False),
        scratch_types=[
            [pltpu.VMEM((CHUNK,), jnp.int32) for _ in range(2)],
            [pltpu.VMEM((CHUNK,), jnp.int32) for _ in range(2)],
            [pltpu.VMEM((CHUNK, D_PAD), jnp.float32) for _ in range(2)],
            [pltpu.VMEM((CHUNK, 2 * D_FEAT), jnp.float32) for _ in range(2)],
            [pltpu.VMEM((CHUNK, D_PAD), jnp.float32) for _ in range(2)],
            [pltpu.SemaphoreType.DMA for _ in range(2)],
            [pltpu.SemaphoreType.DMA for _ in range(2)],
            [pltpu.SemaphoreType.DMA for _ in range(2)],
        ],
    )
    def body(emb_hbm, comb_hbm, src_hbm, cidx_hbm, out_hbm,
             wi, ci, wbuf, cbuf, obuf, si, sg, so):
        wid = lax.axis_index("s") * NC + lax.axis_index("c")
        base0 = wid * tok_per_w

        def idx_copies(c, s):
            base = base0 + c * CHUNK
            return (
                pltpu.make_async_copy(src_hbm.at[pl.ds(base, CHUNK)], wi[s], si[s]),
                pltpu.make_async_copy(cidx_hbm.at[pl.ds(base, CHUNK)], ci[s], si[s]),
            )

        def gather_copies(s):
            return (
                pltpu.make_async_copy(emb_hbm.at[wi[s]], wbuf[s], sg[s]),
                pltpu.make_async_copy(comb_hbm.at[ci[s]], cbuf[s], sg[s]),
            )

        def out_copy(c, s):
            base = base0 + c * CHUNK
            return pltpu.make_async_copy(obuf[s], out_hbm.at[pl.ds(base, CHUNK)], so[s])

        def start(c, s):
            for cp in idx_copies(c, s):
                cp.start()

        def mid(c, s):
            for cp in idx_copies(c, s):
                cp.wait()
            for cp in gather_copies(s):
                cp.start()

        UNROLL = 8

        def assemble_one(s):
            def assemble(g, carry):
                j0 = g * UNROLL
                for u in range(UNROLL):
                    j = j0 + u
                    for k in range(D_WORD // 16):
                        obuf[s][j, pl.ds(16 * k, 16)] = wbuf[s][j, pl.ds(16 * k, 16)]
                    obuf[s][j, pl.ds(D_WORD, 16)] = cbuf[s][j, pl.ds(0, 16)]
                    obuf[s][j, pl.ds(D_WORD + D_FEAT, 16)] = cbuf[s][j, pl.ds(D_FEAT, 16)]
                return carry

            lax.fori_loop(0, CHUNK // UNROLL, assemble, 0)

        def step(i, b):
            mid(i + 1, 1 - b)
            for cp in gather_copies(b):
                cp.wait()

            @pl.when(i < n_chunks - 2)
            def _():
                start(i + 2, b)

            @pl.when(i >= 2)
            def _():
                out_copy(i, b).wait()

            assemble_one(b)
            out_copy(i, b).start()

        assert n_chunks % 2 == 0 and n_chunks >= 4

        start(0, 0)
        start(1, 1)
        mid(0, 0)

        def pair(p, carry):
            for b in range(2):
                step(2 * p + b, b)
            return carry

        lax.fori_loop(0, (n_chunks - 2) // 2, pair, 0)

        step(n_chunks - 2, 0)

        c = n_chunks - 1
        for cp in gather_copies(1):
            cp.wait()
        out_copy(c, 1).wait()
        assemble_one(1)
        out_copy(c, 1).start()
        out_copy(c, 1).wait()
        out_copy(c - 1, 0).wait()

    return body(emb_lut, comb_table, src, cidx)


def kernel(emb_lut, pos_table, ner_table, source, pos_idx, ner_idx):
    B, L = source.shape
    N = B * L
    V = emb_lut.shape[0]
    n_ner = ner_table.shape[0]
    emb128 = jnp.pad(emb_lut, ((0, 0), (0, D_WORD)))
    src = source.reshape(N).astype(jnp.int32)
    cidx = pos_idx.reshape(N).astype(jnp.int32) * n_ner + ner_idx.reshape(N).astype(jnp.int32)
    comb = jnp.concatenate(
        [jnp.repeat(pos_table, n_ner, axis=0), jnp.tile(ner_table, (pos_table.shape[0], 1))],
        axis=1,
    )
    out = _embed(emb128, comb, src, cidx)
    return out[:, :D_OUT].reshape(B, L, D_OUT)


# speedup vs baseline: 2.0008x; 1.0049x over previous
import functools

import jax
import jax.numpy as jnp
from jax import lax
from jax.experimental import pallas as pl
from jax.experimental.pallas import tpu as pltpu
from jax.experimental.pallas import tpu_sc as plsc

D_WORD = 64
D_FEAT = 16
D_OUT = 96
D_PAD = 128
CHUNK = 256


def _embed(emb_lut, comb_table, src, cidx):
    N = src.shape[0]
    info = plsc.get_sparse_core_info()
    NC, NS = info.num_cores, info.num_subcores
    NW = NC * NS
    assert N % NW == 0
    tok_per_w = N // NW
    assert tok_per_w % CHUNK == 0
    n_chunks = tok_per_w // CHUNK

    mesh = plsc.VectorSubcoreMesh(core_axis_name="c", subcore_axis_name="s")

    @functools.partial(
        pl.kernel,
        out_type=jax.ShapeDtypeStruct((N, D_PAD), jnp.float32),
        mesh=mesh,
        compiler_params=pltpu.CompilerParams(use_tc_tiling_on_sc=False),
        scratch_types=[
            [pltpu.VMEM((CHUNK,), jnp.int32) for _ in range(2)],
            [pltpu.VMEM((CHUNK,), jnp.int32) for _ in range(2)],
            [pltpu.VMEM((CHUNK, 2 * D_FEAT), jnp.float32) for _ in range(2)],
            [pltpu.VMEM((CHUNK, D_PAD), jnp.float32) for _ in range(2)],
            [pltpu.SemaphoreType.DMA for _ in range(2)],
            [pltpu.SemaphoreType.DMA for _ in range(2)],
            [pltpu.SemaphoreType.DMA for _ in range(2)],
        ],
    )
    def body(emb_hbm, comb_hbm, src_hbm, cidx_hbm, out_hbm,
             wi, ci, cbuf, obuf, si, sg, so):
        wid = lax.axis_index("s") * NC + lax.axis_index("c")
        base0 = wid * tok_per_w

        def idx_copies(c, s):
            base = base0 + c * CHUNK
            return (
                pltpu.make_async_copy(src_hbm.at[pl.ds(base, CHUNK)], wi[s], si[s]),
                pltpu.make_async_copy(cidx_hbm.at[pl.ds(base, CHUNK)], ci[s], si[s]),
            )

        def gather_copies(s):
            return (
                pltpu.make_async_copy(emb_hbm.at[wi[s]], obuf[s], sg[s]),
                pltpu.make_async_copy(comb_hbm.at[ci[s]], cbuf[s], sg[s]),
            )

        def out_copy(c, s):
            base = base0 + c * CHUNK
            return pltpu.make_async_copy(obuf[s], out_hbm.at[pl.ds(base, CHUNK)], so[s])

        def start(c, s):
            for cp in idx_copies(c, s):
                cp.start()

        def mid(c, s):
            for cp in idx_copies(c, s):
                cp.wait()
            for cp in gather_copies(s):
                cp.start()

        UNROLL = 8

        def assemble_one(s):
            def assemble(g, carry):
                j0 = g * UNROLL
                for u in range(UNROLL):
                    j = j0 + u
                    obuf[s][j, pl.ds(D_WORD, 16)] = cbuf[s][j, pl.ds(0, 16)]
                    obuf[s][j, pl.ds(D_WORD + D_FEAT, 16)] = cbuf[s][j, pl.ds(D_FEAT, 16)]
                return carry

            lax.fori_loop(0, CHUNK // UNROLL, assemble, 0)

        def step(i, b):
            @pl.when(i >= 1)
            def _():
                out_copy(i - 1, 1 - b).wait()

            mid(i + 1, 1 - b)
            for cp in gather_copies(b):
                cp.wait()

            @pl.when(i < n_chunks - 2)
            def _():
                start(i + 2, b)

            assemble_one(b)
            out_copy(i, b).start()

        assert n_chunks % 2 == 0 and n_chunks >= 4

        start(0, 0)
        start(1, 1)
        mid(0, 0)

        def pair(p, carry):
            for b in range(2):
                step(2 * p + b, b)
            return carry

        lax.fori_loop(0, (n_chunks - 2) // 2, pair, 0)

        step(n_chunks - 2, 0)

        c = n_chunks - 1
        for cp in gather_copies(1):
            cp.wait()
        assemble_one(1)
        out_copy(c, 1).start()
        out_copy(c, 1).wait()
        out_copy(c - 1, 0).wait()

    return body(emb_lut, comb_table, src, cidx)


def kernel(emb_lut, pos_table, ner_table, source, pos_idx, ner_idx):
    B, L = source.shape
    N = B * L
    V = emb_lut.shape[0]
    n_ner = ner_table.shape[0]
    emb128 = jnp.pad(emb_lut, ((0, 0), (0, D_WORD)))
    src = source.reshape(N).astype(jnp.int32)
    cidx = pos_idx.reshape(N).astype(jnp.int32) * n_ner + ner_idx.reshape(N).astype(jnp.int32)
    comb = jnp.concatenate(
        [jnp.repeat(pos_table, n_ner, axis=0), jnp.tile(ner_table, (pos_table.shape[0], 1))],
        axis=1,
    )
    out = _embed(emb128, comb, src, cidx)
    return out[:, :D_OUT].reshape(B, L, D_OUT)


# speedup vs baseline: 2.0033x; 1.0012x over previous
import functools

import jax
import jax.numpy as jnp
from jax import lax
from jax.experimental import pallas as pl
from jax.experimental.pallas import tpu as pltpu
from jax.experimental.pallas import tpu_sc as plsc

D_WORD = 64
D_FEAT = 16
D_OUT = 96
D_PAD = 128
CHUNK = 256


def _embed(emb_lut, comb_table, src, cidx):
    N = src.shape[0]
    info = plsc.get_sparse_core_info()
    NC, NS = info.num_cores, info.num_subcores
    NW = NC * NS
    assert N % NW == 0
    tok_per_w = N // NW
    assert tok_per_w % CHUNK == 0
    n_chunks = tok_per_w // CHUNK

    mesh = plsc.VectorSubcoreMesh(core_axis_name="c", subcore_axis_name="s")

    @functools.partial(
        pl.kernel,
        out_type=jax.ShapeDtypeStruct((N, D_PAD), jnp.float32),
        mesh=mesh,
        compiler_params=pltpu.CompilerParams(use_tc_tiling_on_sc=False),
        scratch_types=[
            [pltpu.VMEM((CHUNK,), jnp.int32) for _ in range(2)],
            [pltpu.VMEM((CHUNK,), jnp.int32) for _ in range(2)],
            [pltpu.VMEM((CHUNK, 2 * D_FEAT), jnp.float32) for _ in range(2)],
            [pltpu.VMEM((CHUNK, D_PAD), jnp.float32) for _ in range(2)],
            [pltpu.SemaphoreType.DMA for _ in range(2)],
            [pltpu.SemaphoreType.DMA for _ in range(2)],
            [pltpu.SemaphoreType.DMA for _ in range(2)],
        ],
    )
    def body(emb_hbm, comb_hbm, src_hbm, cidx_hbm, out_hbm,
             wi, ci, cbuf, obuf, si, sg, so):
        wid = lax.axis_index("s") * NC + lax.axis_index("c")
        base0 = wid * tok_per_w

        def idx_copies(c, s):
            base = base0 + c * CHUNK
            return (
                pltpu.make_async_copy(src_hbm.at[pl.ds(base, CHUNK)], wi[s], si[s]),
                pltpu.make_async_copy(cidx_hbm.at[pl.ds(base, CHUNK)], ci[s], si[s]),
            )

        def gather_copies(s):
            return (
                pltpu.make_async_copy(emb_hbm.at[wi[s]], obuf[s], sg[s]),
                pltpu.make_async_copy(comb_hbm.at[ci[s]], cbuf[s], sg[s]),
            )

        def out_copy(c, s):
            base = base0 + c * CHUNK
            return pltpu.make_async_copy(obuf[s], out_hbm.at[pl.ds(base, CHUNK)], so[s])

        def start(c, s):
            for cp in idx_copies(c, s):
                cp.start()

        def mid(c, s):
            for cp in idx_copies(c, s):
                cp.wait()
            for cp in gather_copies(s):
                cp.start()

        UNROLL = 8

        def assemble_one(s):
            def assemble(g, carry):
                j0 = g * UNROLL
                for u in range(UNROLL):
                    j = j0 + u
                    obuf[s][j, pl.ds(D_WORD, 16)] = cbuf[s][j, pl.ds(0, 16)]
                    obuf[s][j, pl.ds(D_WORD + D_FEAT, 16)] = cbuf[s][j, pl.ds(D_FEAT, 16)]
                return carry

            lax.fori_loop(0, CHUNK // UNROLL, assemble, 0)

        def step(i, b):
            @pl.when(i >= 1)
            def _():
                out_copy(i - 1, 1 - b).wait()

            mid(i + 1, 1 - b)
            for cp in gather_copies(b):
                cp.wait()

            @pl.when(i < n_chunks - 2)
            def _():
                start(i + 2, b)

            assemble_one(b)
            out_copy(i, b).start()

        assert n_chunks % 2 == 0 and n_chunks >= 4

        start(0, 0)
        start(1, 1)
        mid(0, 0)

        def pair(p, carry):
            for b in range(2):
                step(2 * p + b, b)
            return carry

        lax.fori_loop(0, (n_chunks - 2) // 2, pair, 0)

        step(n_chunks - 2, 0)

        c = n_chunks - 1
        for cp in gather_copies(1):
            cp.wait()
        assemble_one(1)
        out_copy(c, 1).start()
        out_copy(c, 1).wait()
        out_copy(c - 1, 0).wait()

    return body(emb_lut, comb_table, src, cidx)


def kernel(emb_lut, pos_table, ner_table, source, pos_idx, ner_idx):
    B, L = source.shape
    N = B * L
    n_ner = ner_table.shape[0]
    emb128 = jnp.pad(emb_lut, ((0, 0), (0, D_WORD)))
    src = source.reshape(N).astype(jnp.int32)
    cidx = pos_idx.reshape(N).astype(jnp.int32) * n_ner + ner_idx.reshape(N).astype(jnp.int32)
    comb = jnp.concatenate(
        [jnp.repeat(pos_table, n_ner, axis=0), jnp.tile(ner_table, (pos_table.shape[0], 1))],
        axis=1,
    )
    out = _embed(emb128, comb, src, cidx)
    return out[:, :D_OUT].reshape(B, L, D_OUT)
